# baseline (speedup 1.0000x reference)
import functools

import jax
import jax.numpy as jnp
from jax import lax
from jax.experimental import pallas as pl
from jax.experimental.pallas import tpu as pltpu
from jax.experimental.pallas import tpu_sc as plsc

_LANES = 16
_BLK = 256
_S_TC = 5120


def _softmax_rows(logits):
    m = jnp.max(logits, axis=1, keepdims=True)
    e = jnp.exp(logits - m)
    return e / jnp.sum(e, axis=1, keepdims=True)


def _fused_body(x_ref, w_ref, e_ref, o_ref):
    logits = jax.lax.dot_general(
        x_ref[...], w_ref[...], (((1,), (1,)), ((), ())),
        preferred_element_type=jnp.float32)
    p = _softmax_rows(logits)
    num = e_ref.shape[1]
    acc = p[:, 0:1] * e_ref[:, 0, :]
    for n in range(1, num):
        acc = acc + p[:, n:n + 1] * e_ref[:, n, :]
    o_ref[...] = acc


def _tc_fused(x, experts, W, S):
    B, D = x.shape
    NUM = W.shape[0]
    return pl.pallas_call(
        _fused_body,
        grid=(S // _BLK,),
        in_specs=[
            pl.BlockSpec((_BLK, D), lambda i: (i, 0)),
            pl.BlockSpec((NUM, D), lambda i: (0, 0)),
            pl.BlockSpec((_BLK, NUM, D), lambda i: (i, 0, 0)),
        ],
        out_specs=pl.BlockSpec((_BLK, D), lambda i: (i, 0)),
        out_shape=jax.ShapeDtypeStruct((B, D), jnp.float32),
    )(x, W, experts)


def _weights_body(x_ref, w_ref, p_ref):
    logits = jax.lax.dot_general(
        x_ref[...], w_ref[...], (((1,), (1,)), ((), ())),
        preferred_element_type=jnp.float32)
    p_ref[...] = _softmax_rows(logits)


def _gate_weights(x, W, row0, nrows):
    B, D = x.shape
    NUM = W.shape[0]
    off = row0 // _BLK
    return pl.pallas_call(
        _weights_body,
        grid=(nrows // _BLK,),
        in_specs=[
            pl.BlockSpec((_BLK, D), lambda i: (i + off, 0)),
            pl.BlockSpec((NUM, D), lambda i: (0, 0)),
        ],
        out_specs=pl.BlockSpec((_BLK, NUM), lambda i: (i, 0)),
        out_shape=jax.ShapeDtypeStruct((nrows, NUM), jnp.float32),
    )(x, W)


def _sc_pool(p_flat, experts, row0, nrows, T=6):
    B, NUM, D = experts.shape
    NC, NS = 2, 16
    NW = NC * NS
    bpw = nrows // NW
    nchunks = bpw // T
    mesh = plsc.VectorSubcoreMesh(core_axis_name="c", subcore_axis_name="s")

    @functools.partial(
        pl.kernel,
        out_type=jax.ShapeDtypeStruct((nrows * D,), jnp.float32),
        mesh=mesh,
        scratch_types=[
            pltpu.VMEM((bpw * NUM,), jnp.float32),
            pltpu.VMEM((T, NUM, D), jnp.float32),
            pltpu.VMEM((T, NUM, D), jnp.float32),
            pltpu.VMEM((T * D,), jnp.float32),
            pltpu.VMEM((T * D,), jnp.float32),
            pltpu.SemaphoreType.DMA,
            pltpu.SemaphoreType.DMA,
            pltpu.SemaphoreType.DMA,
            pltpu.SemaphoreType.DMA,
        ],
    )
    def k(p_hbm, e_hbm, o_hbm, p_v, ea, eb, oa, ob, sa, sb, soa, sob):
        wid = lax.axis_index("s") * NC + lax.axis_index("c")
        lbase = wid * bpw
        gbase = row0 + lbase
        pltpu.sync_copy(p_hbm.at[pl.ds(lbase * NUM, bpw * NUM)], p_v)

        ebufs = ((ea, sa), (eb, sb))
        obufs = ((oa, soa), (ob, sob))

        for b in range(2):
            ebuf, sem = ebufs[b]
            pltpu.async_copy(e_hbm.at[pl.ds(gbase + b * T, T)], ebuf, sem)

        def compute_chunk(c, ebuf, obuf):
            for tp in range(T // 2):
                wvec = p_v[pl.ds((c * T + tp * 2) * NUM, _LANES)]
                for half in range(2):
                    t = tp * 2 + half
                    w = [wvec[half * NUM + n] for n in range(NUM)]

                    def dbody(d, carry, t=t, w=w):
                        sl = pl.ds(d * _LANES, _LANES)
                        acc = w[0] * ebuf[t, 0, sl]
                        for n in range(1, NUM):
                            acc = acc + w[n] * ebuf[t, n, sl]
                        obuf[pl.ds(t * D + d * _LANES, _LANES)] = acc
                        return carry

                    lax.fori_loop(0, D // _LANES, dbody, 0, unroll=4)

        def pair_body(i, carry):
            for b in range(2):
                ebuf, esem = ebufs[b]
                obuf, osem = obufs[b]
                c = i * 2 + b
                pltpu.make_async_copy(
                    e_hbm.at[pl.ds(gbase + c * T, T)], ebuf, esem).wait()

                @pl.when(i >= 1)
                def _drain_out():
                    pltpu.make_async_copy(
                        obuf, o_hbm.at[pl.ds(lbase * D, T * D)], osem).wait()

                compute_chunk(c, ebuf, obuf)
                pltpu.async_copy(
                    obuf, o_hbm.at[pl.ds((lbase + c * T) * D, T * D)], osem)

                @pl.when(c + 2 < nchunks)
                def _prefetch():
                    pltpu.async_copy(
                        e_hbm.at[pl.ds(gbase + (c + 2) * T, T)], ebuf, esem)

            return carry

        lax.fori_loop(0, nchunks // 2, pair_body, 0)

        for b in range(2):
            obuf, osem = obufs[b]
            pltpu.make_async_copy(
                obuf, o_hbm.at[pl.ds(lbase * D, T * D)], osem).wait()

    return k(p_flat, experts).reshape(nrows, D)


@jax.jit
def kernel(x, experts, W):
    B, D = x.shape
    NUM = W.shape[0]
    n_sc = B - _S_TC
    p_sc = _gate_weights(x, W, _S_TC, n_sc)
    out_sc = _sc_pool(p_sc.reshape(n_sc * NUM), experts, _S_TC, n_sc)
    out_full = _tc_fused(x, experts, W, _S_TC)
    return lax.dynamic_update_slice(out_full, out_sc, (_S_TC, 0))


# speedup vs baseline: 1.4000x; 1.4000x over previous
import functools

import jax
import jax.numpy as jnp
from jax import lax
from jax.experimental import pallas as pl
from jax.experimental.pallas import tpu as pltpu
from jax.experimental.pallas import tpu_sc as plsc

_LANES = 16
_BLK = 512
_S_TC = 5120


def _softmax_rows(logits):
    m = jnp.max(logits, axis=1, keepdims=True)
    e = jnp.exp(logits - m)
    return e / jnp.sum(e, axis=1, keepdims=True)


def _fused_body(x_ref, w_ref, e_ref, o_ref):
    logits = jax.lax.dot_general(
        x_ref[...], w_ref[...], (((1,), (1,)), ((), ())),
        preferred_element_type=jnp.float32)
    p = _softmax_rows(logits)
    num = e_ref.shape[1]
    acc = p[:, 0:1] * e_ref[:, 0, :]
    for n in range(1, num):
        acc = acc + p[:, n:n + 1] * e_ref[:, n, :]
    o_ref[...] = acc


def _tc_fused(x, experts, W, S):
    B, D = x.shape
    NUM = W.shape[0]
    return pl.pallas_call(
        _fused_body,
        grid=(S // _BLK,),
        in_specs=[
            pl.BlockSpec((_BLK, D), lambda i: (i, 0)),
            pl.BlockSpec((NUM, D), lambda i: (0, 0)),
            pl.BlockSpec((_BLK, NUM, D), lambda i: (i, 0, 0)),
        ],
        out_specs=pl.BlockSpec((_BLK, D), lambda i: (i, 0)),
        out_shape=jax.ShapeDtypeStruct((B, D), jnp.float32),
    )(x, W, experts)


def _weights_body(x_ref, w_ref, p_ref):
    logits = jax.lax.dot_general(
        x_ref[...], w_ref[...], (((1,), (1,)), ((), ())),
        preferred_element_type=jnp.float32)
    p_ref[...] = _softmax_rows(logits)


def _gate_weights(x, W, row0, nrows):
    B, D = x.shape
    NUM = W.shape[0]
    off = row0 // _BLK
    return pl.pallas_call(
        _weights_body,
        grid=(nrows // _BLK,),
        in_specs=[
            pl.BlockSpec((_BLK, D), lambda i: (i + off, 0)),
            pl.BlockSpec((NUM, D), lambda i: (0, 0)),
        ],
        out_specs=pl.BlockSpec((_BLK, NUM), lambda i: (i, 0)),
        out_shape=jax.ShapeDtypeStruct((nrows, NUM), jnp.float32),
    )(x, W)


def _sc_pool(p_flat, experts, row0, nrows, T=6):
    B, NUM, D = experts.shape
    NC, NS = 2, 16
    NW = NC * NS
    bpw = nrows // NW
    nchunks = bpw // T
    mesh = plsc.VectorSubcoreMesh(core_axis_name="c", subcore_axis_name="s")

    @functools.partial(
        pl.kernel,
        out_type=jax.ShapeDtypeStruct((nrows * D,), jnp.float32),
        mesh=mesh,
        scratch_types=[
            pltpu.VMEM((bpw * NUM,), jnp.float32),
            pltpu.VMEM((T, NUM, D), jnp.float32),
            pltpu.VMEM((T, NUM, D), jnp.float32),
            pltpu.VMEM((T * D,), jnp.float32),
            pltpu.VMEM((T * D,), jnp.float32),
            pltpu.SemaphoreType.DMA,
            pltpu.SemaphoreType.DMA,
            pltpu.SemaphoreType.DMA,
            pltpu.SemaphoreType.DMA,
        ],
    )
    def k(p_hbm, e_hbm, o_hbm, p_v, ea, eb, oa, ob, sa, sb, soa, sob):
        wid = lax.axis_index("s") * NC + lax.axis_index("c")
        lbase = wid * bpw
        gbase = row0 + lbase
        pltpu.sync_copy(p_hbm.at[pl.ds(lbase * NUM, bpw * NUM)], p_v)

        ebufs = ((ea, sa), (eb, sb))
        obufs = ((oa, soa), (ob, sob))

        for b in range(2):
            ebuf, sem = ebufs[b]
            pltpu.async_copy(e_hbm.at[pl.ds(gbase + b * T, T)], ebuf, sem)

        def compute_chunk(c, ebuf, obuf):
            for tp in range(T // 2):
                wvec = p_v[pl.ds((c * T + tp * 2) * NUM, _LANES)]
                for half in range(2):
                    t = tp * 2 + half
                    w = [wvec[half * NUM + n] for n in range(NUM)]

                    def dbody(d, carry, t=t, w=w):
                        sl = pl.ds(d * _LANES, _LANES)
                        acc = w[0] * ebuf[t, 0, sl]
                        for n in range(1, NUM):
                            acc = acc + w[n] * ebuf[t, n, sl]
                        obuf[pl.ds(t * D + d * _LANES, _LANES)] = acc
                        return carry

                    lax.fori_loop(0, D // _LANES, dbody, 0, unroll=4)

        def pair_body(i, carry):
            for b in range(2):
                ebuf, esem = ebufs[b]
                obuf, osem = obufs[b]
                c = i * 2 + b
                pltpu.make_async_copy(
                    e_hbm.at[pl.ds(gbase + c * T, T)], ebuf, esem).wait()

                @pl.when(i >= 1)
                def _drain_out():
                    pltpu.make_async_copy(
                        obuf, o_hbm.at[pl.ds(lbase * D, T * D)], osem).wait()

                compute_chunk(c, ebuf, obuf)
                pltpu.async_copy(
                    obuf, o_hbm.at[pl.ds((lbase + c * T) * D, T * D)], osem)

                @pl.when(c + 2 < nchunks)
                def _prefetch():
                    pltpu.async_copy(
                        e_hbm.at[pl.ds(gbase + (c + 2) * T, T)], ebuf, esem)

            return carry

        lax.fori_loop(0, nchunks // 2, pair_body, 0)

        for b in range(2):
            obuf, osem = obufs[b]
            pltpu.make_async_copy(
                obuf, o_hbm.at[pl.ds(lbase * D, T * D)], osem).wait()

    return k(p_flat, experts).reshape(nrows, D)


@jax.jit
def kernel(x, experts, W):
    B, D = x.shape
    return _tc_fused(x, experts, W, B)
